# Initial kernel scaffold; baseline (speedup 1.0000x reference)
import jax
import jax.numpy as jnp
from jax.experimental import pallas as pl


def kernel(start, end):
    raise NotImplementedError("write your pallas kernel here")


# speedup vs baseline: 1.7327x; 1.7327x over previous
import numpy as np
import jax
import jax.numpy as jnp
from jax.experimental import pallas as pl

_POOLING_COUNTS = [15, 8, 8]
_N = 64


def _mask2d_np():
    mask = np.zeros((_N, _N), dtype=bool)
    mask[np.arange(_N), np.arange(_N)] = True
    stride, offset = 1, 0
    for c in _POOLING_COUNTS:
        for _ in range(c):
            offset += stride
            i = np.arange(0, _N - offset)
            mask[i, i + offset] = True
        stride *= 2
    return mask


def _body(start_ref, end_ref, mask_ref, out_ref):
    k = pl.program_id(1)
    half = pl.num_programs(1) // 2
    m = mask_ref[...]

    @pl.when(k < half)
    def _():
        s = start_ref[...]
        out_ref[...] = jnp.where(m[None, :, :] != 0.0, s[:, :, None], 0.0)

    @pl.when(k >= half)
    def _():
        e = end_ref[...]
        out_ref[...] = jnp.where(m[None, :, :] != 0.0, e[:, None, :], 0.0)


def kernel(start, end):
    B, D, N = start.shape
    CB = 128
    half = D // CB
    mask_np = _mask2d_np()
    mask_f = jnp.asarray(mask_np, dtype=jnp.float32)
    map2d = pl.pallas_call(
        _body,
        grid=(B, 2 * half),
        in_specs=[
            pl.BlockSpec((None, CB, N), lambda b, k: (b, jnp.minimum(k, half - 1), 0)),
            pl.BlockSpec((None, CB, N), lambda b, k: (b, jnp.maximum(k - half, 0), 0)),
            pl.BlockSpec((N, N), lambda b, k: (0, 0)),
        ],
        out_specs=pl.BlockSpec((None, CB, N, N), lambda b, k: (b, k, 0, 0)),
        out_shape=jax.ShapeDtypeStruct((B, 2 * D, N, N), start.dtype),
    )(start, end, mask_f)
    return map2d, jnp.asarray(mask_np)


# speedup vs baseline: 2.5879x; 1.4935x over previous
import numpy as np
import jax
import jax.numpy as jnp
from jax.experimental import pallas as pl

_POOLING_COUNTS = [15, 8, 8]
_N = 64


def _mask2d_np():
    mask = np.zeros((_N, _N), dtype=bool)
    mask[np.arange(_N), np.arange(_N)] = True
    stride, offset = 1, 0
    for c in _POOLING_COUNTS:
        for _ in range(c):
            offset += stride
            i = np.arange(0, _N - offset)
            mask[i, i + offset] = True
        stride *= 2
    return mask


def _body(start_ref, end_ref, m1_ref, m2_ref, out_ref):
    k = pl.program_id(1)
    half = pl.num_programs(1) // 2

    @pl.when(k < half)
    def _():
        s = start_ref[...]
        out_ref[...] = jnp.dot(s, m1_ref[...], preferred_element_type=jnp.float32)

    @pl.when(k >= half)
    def _():
        e = end_ref[...]
        out_ref[...] = jnp.dot(e, m2_ref[...], preferred_element_type=jnp.float32)


def kernel(start, end):
    B, D, N = start.shape
    CB = 128
    half = D // CB
    mask_np = _mask2d_np()
    ii, jj = np.nonzero(mask_np)
    m1_np = np.zeros((N, N * N), dtype=np.float32)
    m1_np[ii, ii * N + jj] = 1.0
    m2_np = np.zeros((N, N * N), dtype=np.float32)
    m2_np[jj, ii * N + jj] = 1.0
    m1 = jnp.asarray(m1_np)
    m2 = jnp.asarray(m2_np)
    flat = pl.pallas_call(
        _body,
        grid=(B, 2 * half),
        in_specs=[
            pl.BlockSpec((None, CB, N), lambda b, k: (b, jnp.minimum(k, half - 1), 0)),
            pl.BlockSpec((None, CB, N), lambda b, k: (b, jnp.maximum(k - half, 0), 0)),
            pl.BlockSpec((N, N * N), lambda b, k: (0, 0)),
            pl.BlockSpec((N, N * N), lambda b, k: (0, 0)),
        ],
        out_specs=pl.BlockSpec((None, CB, N * N), lambda b, k: (b, k, 0)),
        out_shape=jax.ShapeDtypeStruct((B, 2 * D, N * N), start.dtype),
    )(start, end, m1, m2)
    return flat.reshape(B, 2 * D, N, N), jnp.asarray(mask_np)


# speedup vs baseline: 2.8354x; 1.0957x over previous
import numpy as np
import jax
import jax.numpy as jnp
from jax.experimental import pallas as pl

_POOLING_COUNTS = [15, 8, 8]
_N = 64


def _mask2d_np():
    mask = np.zeros((_N, _N), dtype=bool)
    mask[np.arange(_N), np.arange(_N)] = True
    stride, offset = 1, 0
    for c in _POOLING_COUNTS:
        for _ in range(c):
            offset += stride
            i = np.arange(0, _N - offset)
            mask[i, i + offset] = True
        stride *= 2
    return mask


def _body(start_ref, end_ref, m1_ref, m2_ref, out_ref):
    k = pl.program_id(1)
    half = pl.num_programs(1) // 2

    @pl.when(k < half)
    def _():
        s = start_ref[...]
        out_ref[...] = jnp.dot(s, m1_ref[...], preferred_element_type=jnp.float32)

    @pl.when(k >= half)
    def _():
        e = end_ref[...]
        out_ref[...] = jnp.dot(e, m2_ref[...], preferred_element_type=jnp.float32)


def kernel(start, end):
    B, D, N = start.shape
    CB = 256
    half = D // CB
    mask_np = _mask2d_np()
    ii, jj = np.nonzero(mask_np)
    m1_np = np.zeros((N, N * N), dtype=np.float32)
    m1_np[ii, ii * N + jj] = 1.0
    m2_np = np.zeros((N, N * N), dtype=np.float32)
    m2_np[jj, ii * N + jj] = 1.0
    m1 = jnp.asarray(m1_np)
    m2 = jnp.asarray(m2_np)
    flat = pl.pallas_call(
        _body,
        grid=(B, 2 * half),
        in_specs=[
            pl.BlockSpec((None, CB, N), lambda b, k: (b, jnp.minimum(k, half - 1), 0)),
            pl.BlockSpec((None, CB, N), lambda b, k: (b, jnp.maximum(k - half, 0), 0)),
            pl.BlockSpec((N, N * N), lambda b, k: (0, 0)),
            pl.BlockSpec((N, N * N), lambda b, k: (0, 0)),
        ],
        out_specs=pl.BlockSpec((None, CB, N * N), lambda b, k: (b, k, 0)),
        out_shape=jax.ShapeDtypeStruct((B, 2 * D, N * N), start.dtype),
    )(start, end, m1, m2)
    return flat.reshape(B, 2 * D, N, N), jnp.asarray(mask_np)


# speedup vs baseline: 2.9354x; 1.0353x over previous
import numpy as np
import jax
import jax.numpy as jnp
from jax.experimental import pallas as pl

_POOLING_COUNTS = [15, 8, 8]
_N = 64


def _mask2d_np():
    mask = np.zeros((_N, _N), dtype=bool)
    mask[np.arange(_N), np.arange(_N)] = True
    stride, offset = 1, 0
    for c in _POOLING_COUNTS:
        for _ in range(c):
            offset += stride
            i = np.arange(0, _N - offset)
            mask[i, i + offset] = True
        stride *= 2
    return mask


def _body(start_ref, end_ref, m1_ref, m2_ref, out_ref):
    k = pl.program_id(1)
    half = pl.num_programs(1) // 2

    @pl.when(k < half)
    def _():
        s = start_ref[...]
        out_ref[...] = jnp.dot(s, m1_ref[...], preferred_element_type=jnp.float32)

    @pl.when(k >= half)
    def _():
        e = end_ref[...]
        out_ref[...] = jnp.dot(e, m2_ref[...], preferred_element_type=jnp.float32)


def kernel(start, end):
    B, D, N = start.shape
    CB = 512
    half = D // CB
    mask_np = _mask2d_np()
    ii, jj = np.nonzero(mask_np)
    m1_np = np.zeros((N, N * N), dtype=np.float32)
    m1_np[ii, ii * N + jj] = 1.0
    m2_np = np.zeros((N, N * N), dtype=np.float32)
    m2_np[jj, ii * N + jj] = 1.0
    m1 = jnp.asarray(m1_np)
    m2 = jnp.asarray(m2_np)
    flat = pl.pallas_call(
        _body,
        grid=(B, 2 * half),
        in_specs=[
            pl.BlockSpec((None, CB, N), lambda b, k: (b, jnp.minimum(k, half - 1), 0)),
            pl.BlockSpec((None, CB, N), lambda b, k: (b, jnp.maximum(k - half, 0), 0)),
            pl.BlockSpec((N, N * N), lambda b, k: (0, 0)),
            pl.BlockSpec((N, N * N), lambda b, k: (0, 0)),
        ],
        out_specs=pl.BlockSpec((None, CB, N * N), lambda b, k: (b, k, 0)),
        out_shape=jax.ShapeDtypeStruct((B, 2 * D, N * N), start.dtype),
    )(start, end, m1, m2)
    return flat.reshape(B, 2 * D, N, N), jnp.asarray(mask_np)
